# Initial kernel scaffold; baseline (speedup 1.0000x reference)
import jax
import jax.numpy as jnp
from jax.experimental import pallas as pl


def kernel(center_words, context_words, negative_words, W_in, W_out):
    raise NotImplementedError("write your pallas kernel here")


# speedup vs baseline: 3.9562x; 3.9562x over previous
import functools

import jax
import jax.numpy as jnp
from jax import lax
from jax.experimental import pallas as pl
from jax.experimental.pallas import tpu as pltpu
from jax.experimental.pallas import tpu_sc as plsc

D = 64
K = 20
NC = 2
NS = 16
NW = NC * NS
S = 32
L = 16
G = S // L
NIDX_COLS = 128
NIDX_ROWS = S * K // NIDX_COLS


def _body(c_hbm, x_hbm, n_hbm, win, wout, pos_out, neg_out,
          cidx, xidx, nidx, crow, xrow, nrow, posv, negv, sem,
          *, bw, nchunk):
    wid = lax.axis_index("s") * NC + lax.axis_index("c")
    base = wid * bw
    iota = lax.iota(jnp.int32, L)
    rowc = [g * L + iota for g in range(G)]
    rown = [[(g * L * K + k) + K * iota for k in range(K)] for g in range(G)]

    def chunk_body(i, carry):
        cb = base + i * S
        pltpu.sync_copy(c_hbm.at[pl.ds(cb, S)], cidx)
        pltpu.sync_copy(x_hbm.at[pl.ds(cb, S)], xidx)
        pltpu.sync_copy(n_hbm.at[pl.ds(cb * K, S * K)], nidx)
        cps = [pltpu.async_copy(win.at[cidx], crow, sem),
               pltpu.async_copy(wout.at[xidx], xrow, sem)]
        for j in range(NIDX_ROWS):
            cps.append(pltpu.async_copy(
                wout.at[nidx.at[pl.ds(j * NIDX_COLS, NIDX_COLS)]],
                nrow.at[pl.ds(j * NIDX_COLS, NIDX_COLS)],
                sem))
        for cp in cps:
            cp.wait()

        for g in range(G):
            def dstep(d, accs, g=g):
                acc_pos, acc_neg = accs
                col = jnp.full((L,), d, dtype=jnp.int32)
                c = plsc.load_gather(crow, [rowc[g], col])
                x = plsc.load_gather(xrow, [rowc[g], col])
                acc_pos = acc_pos + c * x
                new_neg = []
                for k in range(K):
                    nv = plsc.load_gather(nrow, [rown[g][k], col])
                    new_neg.append(acc_neg[k] + c * nv)
                return acc_pos, tuple(new_neg)

            z = jnp.zeros((L,), jnp.float32)
            acc_pos, acc_neg = lax.fori_loop(0, D, dstep, (z, (z,) * K))
            posv[pl.ds(g * L, L)] = acc_pos
            for k in range(K):
                plsc.store_scatter(negv, [rown[g][k]], acc_neg[k])
        pltpu.sync_copy(posv, pos_out.at[pl.ds(cb, S)])
        pltpu.sync_copy(negv, neg_out.at[pl.ds(cb * K, S * K)])
        return carry

    lax.fori_loop(0, nchunk, chunk_body, 0)


def kernel(center_words, context_words, negative_words, W_in, W_out):
    b = center_words.shape[0]
    bw = b // NW
    nchunk = bw // S
    mesh = plsc.VectorSubcoreMesh(core_axis_name="c", subcore_axis_name="s")
    k = pl.kernel(
        functools.partial(_body, bw=bw, nchunk=nchunk),
        out_type=(jax.ShapeDtypeStruct((b,), jnp.float32),
                  jax.ShapeDtypeStruct((b * K,), jnp.float32)),
        mesh=mesh,
        compiler_params=pltpu.CompilerParams(
            needs_layout_passes=False, use_tc_tiling_on_sc=False),
        scratch_types=[
            pltpu.VMEM((S,), jnp.int32),
            pltpu.VMEM((S,), jnp.int32),
            pltpu.VMEM((S * K,), jnp.int32),
            pltpu.VMEM((S, D), jnp.float32),
            pltpu.VMEM((S, D), jnp.float32),
            pltpu.VMEM((S * K, D), jnp.float32),
            pltpu.VMEM((S,), jnp.float32),
            pltpu.VMEM((S * K,), jnp.float32),
            pltpu.SemaphoreType.DMA,
        ],
    )
    nflat = negative_words.astype(jnp.int32).reshape(-1)
    pos, neg = k(center_words.astype(jnp.int32),
                 context_words.astype(jnp.int32),
                 nflat, W_in, W_out)
    return pos, neg.reshape(b, K)


# speedup vs baseline: 4.0175x; 1.0155x over previous
import functools

import jax
import jax.numpy as jnp
from jax import lax
from jax.experimental import pallas as pl
from jax.experimental.pallas import tpu as pltpu
from jax.experimental.pallas import tpu_sc as plsc

D = 64
K = 20
NC = 2
NS = 16
NW = NC * NS
S = 32
L = 16
G = S // L
NIDX_COLS = 128
NIDX_ROWS = S * K // NIDX_COLS


def _body(c_hbm, x_hbm, n_hbm, win, wout, pos_out, neg_out,
          cidx, xidx, nidx, crow0, crow1, xrow0, xrow1, nrow0, nrow1,
          posv, negv, sem0, sem1, *, bw, nchunk):
    wid = lax.axis_index("s") * NC + lax.axis_index("c")
    base = wid * bw
    iota = lax.iota(jnp.int32, L)
    rowc = [g * L + iota for g in range(G)]
    rown = [[(g * L * K + k) + K * iota for k in range(K)] for g in range(G)]
    crow = (crow0, crow1)
    xrow = (xrow0, xrow1)
    nrow = (nrow0, nrow1)
    sems = (sem0, sem1)

    pltpu.sync_copy(c_hbm.at[pl.ds(base, bw)], cidx)
    pltpu.sync_copy(x_hbm.at[pl.ds(base, bw)], xidx)
    pltpu.sync_copy(n_hbm.at[pl.ds(base * K, bw * K)], nidx)

    def issue(i, buf):
        pltpu.async_copy(win.at[cidx.at[pl.ds(i * S, S)]], crow[buf], sems[buf])
        pltpu.async_copy(wout.at[xidx.at[pl.ds(i * S, S)]], xrow[buf], sems[buf])
        for j in range(NIDX_ROWS):
            pltpu.async_copy(
                wout.at[nidx.at[pl.ds(i * S * K + j * NIDX_COLS, NIDX_COLS)]],
                nrow[buf].at[pl.ds(j * NIDX_COLS, NIDX_COLS)], sems[buf])

    def drain(buf):
        pltpu.make_async_copy(
            win.at[cidx.at[pl.ds(0, S)]], crow[buf], sems[buf]).wait()
        pltpu.make_async_copy(
            wout.at[xidx.at[pl.ds(0, S)]], xrow[buf], sems[buf]).wait()
        for j in range(NIDX_ROWS):
            pltpu.make_async_copy(
                wout.at[nidx.at[pl.ds(j * NIDX_COLS, NIDX_COLS)]],
                nrow[buf].at[pl.ds(j * NIDX_COLS, NIDX_COLS)],
                sems[buf]).wait()

    def compute(i, buf):
        for g in range(G):
            def dstep(d, accs, g=g):
                acc_pos, acc_neg = accs
                col = jnp.full((L,), d, dtype=jnp.int32)
                c = plsc.load_gather(crow[buf], [rowc[g], col])
                x = plsc.load_gather(xrow[buf], [rowc[g], col])
                acc_pos = acc_pos + c * x
                new_neg = [
                    acc_neg[k] + c * plsc.load_gather(nrow[buf],
                                                      [rown[g][k], col])
                    for k in range(K)]
                return acc_pos, tuple(new_neg)

            z = jnp.zeros((L,), jnp.float32)
            acc_pos, acc_neg = lax.fori_loop(0, D, dstep, (z, (z,) * K))
            plsc.store_scatter(posv, [i * S + g * L + iota], acc_pos)
            for k in range(K):
                plsc.store_scatter(negv, [i * (S * K) + rown[g][k]],
                                   acc_neg[k])

    issue(0, 0)

    def pair(p, carry):
        i0 = 2 * p
        issue(i0 + 1, 1)
        drain(0)
        compute(i0, 0)
        issue(jnp.minimum(i0 + 2, nchunk - 1), 0)
        drain(1)
        compute(i0 + 1, 1)
        return carry

    lax.fori_loop(0, nchunk // 2, pair, 0)
    drain(0)
    pltpu.sync_copy(posv, pos_out.at[pl.ds(base, bw)])
    pltpu.sync_copy(negv, neg_out.at[pl.ds(base * K, bw * K)])


def kernel(center_words, context_words, negative_words, W_in, W_out):
    b = center_words.shape[0]
    bw = b // NW
    nchunk = bw // S
    mesh = plsc.VectorSubcoreMesh(core_axis_name="c", subcore_axis_name="s")
    k = pl.kernel(
        functools.partial(_body, bw=bw, nchunk=nchunk),
        out_type=(jax.ShapeDtypeStruct((b,), jnp.float32),
                  jax.ShapeDtypeStruct((b * K,), jnp.float32)),
        mesh=mesh,
        compiler_params=pltpu.CompilerParams(
            needs_layout_passes=False, use_tc_tiling_on_sc=False),
        scratch_types=[
            pltpu.VMEM((bw,), jnp.int32),
            pltpu.VMEM((bw,), jnp.int32),
            pltpu.VMEM((bw * K,), jnp.int32),
            pltpu.VMEM((S, D), jnp.float32),
            pltpu.VMEM((S, D), jnp.float32),
            pltpu.VMEM((S, D), jnp.float32),
            pltpu.VMEM((S, D), jnp.float32),
            pltpu.VMEM((S * K, D), jnp.float32),
            pltpu.VMEM((S * K, D), jnp.float32),
            pltpu.VMEM((bw,), jnp.float32),
            pltpu.VMEM((bw * K,), jnp.float32),
            pltpu.SemaphoreType.DMA,
            pltpu.SemaphoreType.DMA,
        ],
    )
    nflat = negative_words.astype(jnp.int32).reshape(-1)
    pos, neg = k(center_words.astype(jnp.int32),
                 context_words.astype(jnp.int32),
                 nflat, W_in, W_out)
    return pos, neg.reshape(b, K)


# speedup vs baseline: 4.8972x; 1.2190x over previous
import functools

import jax
import jax.numpy as jnp
from jax import lax
from jax.experimental import pallas as pl
from jax.experimental.pallas import tpu as pltpu
from jax.experimental.pallas import tpu_sc as plsc

D = 64
K = 20
NC = 2
NS = 16
NW = NC * NS
S = 32
L = 16
NV = D // L


def _body(c_hbm, x_hbm, n_hbm, win, wout, pos_out, neg_out,
          cidx, xidx, nidx, crow0, crow1, xrow0, xrow1, nrow0, nrow1,
          posv, negv, sem0, sem1, *, bw, nchunk):
    wid = lax.axis_index("s") * NC + lax.axis_index("c")
    base = wid * bw
    iota = lax.iota(jnp.int32, L)
    lane15 = iota == (L - 1)
    crow = (crow0, crow1)
    xrow = (xrow0, xrow1)
    nrow = (nrow0, nrow1)
    sems = (sem0, sem1)

    pltpu.sync_copy(c_hbm.at[pl.ds(base, bw)], cidx)
    pltpu.sync_copy(x_hbm.at[pl.ds(base, bw)], xidx)
    pltpu.sync_copy(n_hbm.at[pl.ds(base * K, bw * K)], nidx)

    def issue(i, buf):
        pltpu.async_copy(win.at[cidx.at[pl.ds(i * S, S)]], crow[buf], sems[buf])
        pltpu.async_copy(wout.at[xidx.at[pl.ds(i * S, S)]], xrow[buf], sems[buf])
        pltpu.async_copy(wout.at[nidx.at[pl.ds(i * S * K, S * K)]], nrow[buf],
                         sems[buf])

    def drain(buf):
        pltpu.make_async_copy(
            win.at[cidx.at[pl.ds(0, S)]], crow[buf], sems[buf]).wait()
        pltpu.make_async_copy(
            wout.at[xidx.at[pl.ds(0, S)]], xrow[buf], sems[buf]).wait()
        pltpu.make_async_copy(
            wout.at[nidx.at[pl.ds(0, S * K)]], nrow[buf], sems[buf]).wait()

    def compute(i, buf):
        def bstep(bb, carry):
            c = [crow[buf][bb, pl.ds(j * L, L)] for j in range(NV)]
            x = [xrow[buf][bb, pl.ds(j * L, L)] for j in range(NV)]
            m = c[0] * x[0]
            for j in range(1, NV):
                m = m + c[j] * x[j]
            cum = plsc.cumsum(m)
            gpos = i * S + bb
            plsc.store_scatter(posv, [jnp.full((L,), gpos, jnp.int32)], cum,
                               mask=lane15)
            for k in range(K):
                n = [nrow[buf][bb * K + k, pl.ds(j * L, L)] for j in range(NV)]
                m = c[0] * n[0]
                for j in range(1, NV):
                    m = m + c[j] * n[j]
                cum = plsc.cumsum(m)
                plsc.store_scatter(
                    negv, [jnp.full((L,), gpos * K + k, jnp.int32)], cum,
                    mask=lane15)
            return carry

        lax.fori_loop(0, S, bstep, 0)

    issue(0, 0)

    def pair(p, carry):
        i0 = 2 * p
        issue(i0 + 1, 1)
        drain(0)
        compute(i0, 0)
        issue(jnp.minimum(i0 + 2, nchunk - 1), 0)
        drain(1)
        compute(i0 + 1, 1)
        return carry

    lax.fori_loop(0, nchunk // 2, pair, 0)
    drain(0)
    pltpu.sync_copy(posv, pos_out.at[pl.ds(base, bw)])
    pltpu.sync_copy(negv, neg_out.at[pl.ds(base * K, bw * K)])


def kernel(center_words, context_words, negative_words, W_in, W_out):
    b = center_words.shape[0]
    bw = b // NW
    nchunk = bw // S
    mesh = plsc.VectorSubcoreMesh(core_axis_name="c", subcore_axis_name="s")
    k = pl.kernel(
        functools.partial(_body, bw=bw, nchunk=nchunk),
        out_type=(jax.ShapeDtypeStruct((b,), jnp.float32),
                  jax.ShapeDtypeStruct((b * K,), jnp.float32)),
        mesh=mesh,
        compiler_params=pltpu.CompilerParams(
            needs_layout_passes=False, use_tc_tiling_on_sc=False),
        scratch_types=[
            pltpu.VMEM((bw,), jnp.int32),
            pltpu.VMEM((bw,), jnp.int32),
            pltpu.VMEM((bw * K,), jnp.int32),
            pltpu.VMEM((S, D), jnp.float32),
            pltpu.VMEM((S, D), jnp.float32),
            pltpu.VMEM((S, D), jnp.float32),
            pltpu.VMEM((S, D), jnp.float32),
            pltpu.VMEM((S * K, D), jnp.float32),
            pltpu.VMEM((S * K, D), jnp.float32),
            pltpu.VMEM((bw,), jnp.float32),
            pltpu.VMEM((bw * K,), jnp.float32),
            pltpu.SemaphoreType.DMA,
            pltpu.SemaphoreType.DMA,
        ],
    )
    nflat = negative_words.astype(jnp.int32).reshape(-1)
    pos, neg = k(center_words.astype(jnp.int32),
                 context_words.astype(jnp.int32),
                 nflat, W_in, W_out)
    return pos, neg.reshape(b, K)
